# baseline (speedup 1.0000x reference)
import functools

import jax
import jax.numpy as jnp
from jax import lax
from jax.experimental import pallas as pl
from jax.experimental.pallas import tpu as pltpu
from jax.experimental.pallas import tpu_sc as plsc

_R = 32
_D = 4
_A = 64


def _build():
    mesh = plsc.VectorSubcoreMesh(core_axis_name="c", subcore_axis_name="s")

    @functools.partial(
        pl.kernel,
        mesh=mesh,
        out_type=jax.ShapeDtypeStruct((_A,), jnp.float32),
        scratch_types=[
            pltpu.VMEM((16,), jnp.float32),
            pltpu.VMEM((_D, _R), jnp.float32),
            pltpu.VMEM((_A,), jnp.float32),
        ],
    )
    def table_lookup(x_hbm, grid_hbm, q_hbm, out_hbm, x_v, grid_v, row_v):
        @pl.when((lax.axis_index("c") == 0) & (lax.axis_index("s") == 0))
        def _():
            pltpu.sync_copy(x_hbm, x_v)
            pltpu.sync_copy(grid_hbm, grid_v)
            xv = x_v[:]
            flat = None
            for d in range(_D):
                xd = xv[d]
                g_lo = grid_v[d, pl.ds(0, 16)]
                g_hi = grid_v[d, pl.ds(16, 16)]
                cnt = None
                for half in (g_lo, g_hi):
                    for i in range(16):
                        b = (half[i] <= xd).astype(jnp.int32)
                        cnt = b if cnt is None else cnt + b
                flat = cnt if flat is None else flat * _R + cnt
            pltpu.sync_copy(q_hbm.at[flat], row_v)
            pltpu.sync_copy(row_v, out_hbm)

    return table_lookup


_lookup = _build()


def kernel(x, obs_grid, q_values):
    x16 = jnp.zeros((16,), jnp.float32).at[:_D].set(x)
    grid_t = obs_grid.T
    q2 = q_values.reshape(_R ** _D, _A)
    return _lookup(x16, grid_t, q2)


# speedup vs baseline: 1.0415x; 1.0415x over previous
import functools

import jax
import jax.numpy as jnp
from jax.experimental import pallas as pl
from jax.experimental.pallas import tpu as pltpu
from jax.experimental.pallas import tpu_sc as plsc

_R = 32
_D = 4
_A = 64


def _build():
    mesh = plsc.VectorSubcoreMesh(
        core_axis_name="c", subcore_axis_name="s", num_cores=1, num_subcores=1
    )

    @functools.partial(
        pl.kernel,
        mesh=mesh,
        out_type=jax.ShapeDtypeStruct((_A,), jnp.float32),
        scratch_types=[
            pltpu.VMEM((16,), jnp.float32),
            pltpu.VMEM((_D, _R), jnp.float32),
            pltpu.SemaphoreType.DMA,
            pltpu.SemaphoreType.DMA,
        ],
    )
    def table_lookup(x_hbm, grid_hbm, q_hbm, out_hbm, x_v, grid_v, sem_x, sem_g):
        cp_x = pltpu.async_copy(x_hbm, x_v, sem_x)
        cp_g = pltpu.async_copy(grid_hbm, grid_v, sem_g)
        cp_x.wait()
        cp_g.wait()
        xv = x_v[:]
        flat = None
        for d in range(_D):
            xd = xv[d]
            g_lo = grid_v[d, pl.ds(0, 16)]
            g_hi = grid_v[d, pl.ds(16, 16)]
            cnt = None
            for half in (g_lo, g_hi):
                for i in range(16):
                    b = (half[i] <= xd).astype(jnp.int32)
                    cnt = b if cnt is None else cnt + b
            flat = cnt if flat is None else flat * _R + cnt
        pltpu.sync_copy(q_hbm.at[flat], out_hbm)

    return table_lookup


_lookup = _build()


def kernel(x, obs_grid, q_values):
    x16 = jnp.zeros((16,), jnp.float32).at[:_D].set(x)
    grid_t = obs_grid.T
    q2 = q_values.reshape(_R ** _D, _A)
    return _lookup(x16, grid_t, q2)


# speedup vs baseline: 1.1377x; 1.0924x over previous
import functools

import jax
import jax.numpy as jnp
from jax import lax
from jax.experimental import pallas as pl
from jax.experimental.pallas import tpu as pltpu
from jax.experimental.pallas import tpu_sc as plsc

_R = 32
_D = 4
_A = 64


def _build():
    mesh = plsc.VectorSubcoreMesh(
        core_axis_name="c", subcore_axis_name="s", num_cores=1, num_subcores=1
    )

    @functools.partial(
        pl.kernel,
        mesh=mesh,
        out_type=jax.ShapeDtypeStruct((_A,), jnp.float32),
        scratch_types=[
            pltpu.VMEM((16,), jnp.float32),
            pltpu.VMEM((_R * _D,), jnp.float32),
            pltpu.SemaphoreType.DMA,
            pltpu.SemaphoreType.DMA,
        ],
    )
    def table_lookup(x_hbm, grid_hbm, q_hbm, out_hbm, x_v, grid_v, sem_x, sem_g):
        cp_x = pltpu.async_copy(x_hbm, x_v.at[pl.ds(0, _D)], sem_x)
        cp_g = pltpu.async_copy(grid_hbm, grid_v, sem_g)
        cp_x.wait()
        cp_g.wait()
        xv = x_v[:]
        lane = lax.iota(jnp.int32, 16) & 3
        xq = jnp.full((16,), xv[0])
        for d in range(1, _D):
            xq = jnp.where(lane == d, jnp.full((16,), xv[d]), xq)
        w = None
        for k in range(8):
            g = grid_v[pl.ds(16 * k, 16)]
            wk = jnp.where(g <= xq, 1, 0)
            w = wk if w is None else w + wk
        flat = None
        for d in range(_D):
            cnt = w[d] + w[d + 4] + w[d + 8] + w[d + 12]
            flat = cnt if flat is None else flat * _R + cnt
        pltpu.sync_copy(q_hbm.at[flat], out_hbm)

    return table_lookup


_lookup = _build()


def kernel(x, obs_grid, q_values):
    return _lookup(x, obs_grid.reshape(_R * _D), q_values.reshape(_R ** _D, _A))


# speedup vs baseline: 1.2098x; 1.0634x over previous
import functools

import jax
import jax.numpy as jnp
from jax.experimental import pallas as pl
from jax.experimental.pallas import tpu as pltpu
from jax.experimental.pallas import tpu_sc as plsc

_R = 32
_D = 4
_A = 64


def _build():
    mesh = plsc.ScalarSubcoreMesh(axis_name="c", num_cores=1)

    @functools.partial(
        pl.kernel,
        mesh=mesh,
        out_type=jax.ShapeDtypeStruct((_A,), jnp.float32),
        scratch_types=[
            pltpu.SMEM((_D,), jnp.float32),
            pltpu.SMEM((_R * _D,), jnp.float32),
            pltpu.SemaphoreType.DMA,
            pltpu.SemaphoreType.DMA,
        ],
    )
    def table_lookup(x_hbm, grid_hbm, q_hbm, out_hbm, x_s, g_s, sem_x, sem_g):
        cp_x = pltpu.async_copy(x_hbm, x_s, sem_x)
        cp_g = pltpu.async_copy(grid_hbm, g_s, sem_g)
        cp_x.wait()
        cp_g.wait()
        flat = None
        for d in range(_D):
            xd = x_s[d]
            cnt = None
            for i in range(_R):
                b = (g_s[i * _D + d] <= xd).astype(jnp.int32)
                cnt = b if cnt is None else cnt + b
            flat = cnt if flat is None else flat * _R + cnt
        pltpu.sync_copy(q_hbm.at[flat], out_hbm)

    return table_lookup


_lookup = _build()


def kernel(x, obs_grid, q_values):
    return _lookup(x, obs_grid.reshape(_R * _D), q_values.reshape(_R ** _D, _A))
